# speedup vs baseline: 1.9925x; 1.9925x over previous
import math

import jax
import jax.numpy as jnp
from jax import lax
from jax.experimental import pallas as pl
from jax.experimental.pallas import tpu as pltpu

_V = 100000
_N = 1024
_SMOOTH = 0.1
_CONF = 1.0 - _SMOOTH
_S = _SMOOTH / (_V - 2)
_W = _V - 100
_SLOGS = _S * math.log(_S)
_K0 = (_V - 2) * _SLOGS + _CONF * math.log(_CONF)

_BR = 32


def _body(x_ref, t_ref, o_ref, acc_ref):
    j = pl.program_id(0)

    @pl.when(j == 0)
    def _init():
        acc_ref[0] = 0.0
        acc_ref[1] = 0.0
        acc_ref[2] = 0.0
        acc_ref[3] = 0.0
        acc_ref[4] = 0.0

    x = x_ref[...]
    t = t_ref[...]
    col = lax.broadcasted_iota(jnp.int32, x.shape, 1)
    onehot = col == t
    isw = t == _W

    acc_ref[0] += jnp.sum(x)
    acc_ref[1] += jnp.sum(x[:, _W])
    acc_ref[2] += jnp.sum(jnp.where(onehot, x, 0.0))
    acc_ref[3] += jnp.sum(jnp.where(onehot & isw, x, 0.0))
    acc_ref[4] += jnp.sum(jnp.where(isw, 1.0, 0.0))

    @pl.when(j == pl.num_programs(0) - 1)
    def _fin():
        o_ref[0, 0] = (
            _N * _K0
            + acc_ref[4] * _SLOGS
            - _S * acc_ref[0]
            + _S * acc_ref[1]
            + (_S - _CONF) * acc_ref[2]
            - _S * acc_ref[3]
        )


def kernel(output, target):
    t2 = target.reshape(_N, 1)
    out = pl.pallas_call(
        _body,
        grid=(_N // _BR,),
        in_specs=[
            pl.BlockSpec((_BR, _V), lambda j: (j, 0)),
            pl.BlockSpec((_BR, 1), lambda j: (j, 0)),
        ],
        out_specs=pl.BlockSpec(
            (1, 1), lambda j: (0, 0), memory_space=pltpu.SMEM
        ),
        out_shape=jax.ShapeDtypeStruct((1, 1), jnp.float32),
        scratch_shapes=[pltpu.SMEM((8,), jnp.float32)],
    )(output, t2)
    return out[0, 0]
